# Initial kernel scaffold; baseline (speedup 1.0000x reference)
import jax
import jax.numpy as jnp
from jax.experimental import pallas as pl


def kernel(input_ids, W, b):
    raise NotImplementedError("write your pallas kernel here")


# speedup vs baseline: 28.0075x; 28.0075x over previous
import jax
import jax.numpy as jnp
from jax import lax
from jax.experimental import pallas as pl
from jax.experimental.pallas import tpu as pltpu
from jax.experimental.pallas import tpu_sc as plsc

_B = 1024
_L = 200
_C = 2
_V = 100000

_NC = 2
_NS = 16
_RPW = _B // _NS
_G = _RPW // 16


def _bow_kernel(w_hbm, ids_hbm, out_hbm, w_v, ids_v, out_v):
  c = lax.axis_index("c")
  s = lax.axis_index("s")

  pltpu.sync_copy(w_hbm.at[c], w_v)
  pltpu.sync_copy(ids_hbm.at[s], ids_v)

  def body(l, accs):
    new = []
    for g in range(_G):
      idx = ids_v[l, pl.ds(g * 16, 16)]
      val = plsc.load_gather(w_v, [idx])
      new.append(accs[g] + jnp.where(idx != 0, val, 0.0))
    return tuple(new)

  zero = jnp.zeros((16,), jnp.float32)
  accs = lax.fori_loop(0, _L, body, (zero,) * _G)

  for g in range(_G):
    out_v[pl.ds(g * 16, 16)] = accs[g]
  pltpu.sync_copy(out_v, out_hbm.at[c, pl.ds(s * _RPW, _RPW)])


def _make_call():
  mesh = plsc.VectorSubcoreMesh(core_axis_name="c", subcore_axis_name="s")
  return pl.kernel(
      _bow_kernel,
      out_type=jax.ShapeDtypeStruct((_C, _B), jnp.float32),
      mesh=mesh,
      compiler_params=pltpu.CompilerParams(needs_layout_passes=False),
      scratch_types=[
          pltpu.VMEM((_V,), jnp.float32),
          pltpu.VMEM((_L, _RPW), jnp.int32),
          pltpu.VMEM((_RPW,), jnp.float32),
      ],
  )


_call = _make_call()


@jax.jit
def kernel(input_ids, W, b):
  ids = input_ids.astype(jnp.int32)
  ids_r = ids.T.reshape(_L, _NS, _RPW).transpose(1, 0, 2)
  out = _call(W, ids_r)
  return out.T + b[None, :]
